# baseline (speedup 1.0000x reference)
import functools

import jax
import jax.numpy as jnp
from jax import lax
from jax.experimental import pallas as pl
from jax.experimental.pallas import tpu as pltpu
from jax.experimental.pallas import tpu_sc as plsc

_FIELD0 = 1000000
_BLK1 = _FIELD0 // 8
_B = 16384
_D = 16
_NC, _NS = 2, 16
_NW = _NC * _NS
_BPW = _B // _NW
_CH = 128
_NCH = _BPW // _CH
_GPC = _CH // _D


def _body(x0_hbm, x1_hbm, table_hbm, out_hbm,
          xi0_v, xi1_v, blk0_v, blk1_v, dat0_v, dat1_v, out_v, sem):
  wid = lax.axis_index("s") * _NC + lax.axis_index("c")
  base = wid * _BPW

  pltpu.sync_copy(x0_hbm.at[pl.ds(base, _BPW)], xi0_v)
  pltpu.sync_copy(x1_hbm.at[pl.ds(base, _BPW)], xi1_v)
  for c in range(_BPW // _D):
    j, sl = c // _GPC, pl.ds((c % _GPC) * _D, _D)
    blk0_v[j, sl] = lax.shift_right_logical(xi0_v[pl.ds(c * _D, _D)], 3)
    blk1_v[j, sl] = lax.shift_right_logical(xi1_v[pl.ds(c * _D, _D)], 3) + _BLK1

  lane = lax.iota(jnp.int32, _D)

  for j in range(_NCH):
    c0 = pltpu.async_copy(table_hbm.at[blk0_v.at[j]], dat0_v, sem)
    c1 = pltpu.async_copy(table_hbm.at[blk1_v.at[j]], dat1_v, sem)
    c0.wait()
    c1.wait()
    for g in range(_GPC):
      row = pl.ds((j * _GPC + g) * _D, _D)
      off0 = (xi0_v[row] & 7) * _D
      off1 = (xi1_v[row] & 7) * _D
      gr = g * _D + lane
      acc = jnp.zeros((_D,), jnp.float32)
      for l in range(_D):
        a = plsc.load_gather(dat0_v, [gr, off0 + l])
        b = plsc.load_gather(dat1_v, [gr, off1 + l])
        acc = acc + a * b
      e = jnp.exp(-jnp.abs(acc))
      out_v[row] = jnp.where(acc >= 0.0, 1.0 / (1.0 + e), e / (1.0 + e))

  pltpu.sync_copy(out_v, out_hbm.at[pl.ds(base, _BPW)])


@jax.jit
def _run(x0, x1, table_blocks):
  mesh = plsc.VectorSubcoreMesh(core_axis_name="c", subcore_axis_name="s",
                                num_cores=_NC, num_subcores=_NS)
  return pl.kernel(
      _body,
      out_type=jax.ShapeDtypeStruct((_B,), jnp.float32),
      mesh=mesh,
      compiler_params=pltpu.CompilerParams(needs_layout_passes=False,
                                           use_tc_tiling_on_sc=False),
      scratch_types=[
          pltpu.VMEM((_BPW,), jnp.int32),
          pltpu.VMEM((_BPW,), jnp.int32),
          pltpu.VMEM((_NCH, _CH), jnp.int32),
          pltpu.VMEM((_NCH, _CH), jnp.int32),
          pltpu.VMEM((_CH, 8 * _D), jnp.float32),
          pltpu.VMEM((_CH, 8 * _D), jnp.float32),
          pltpu.VMEM((_BPW,), jnp.float32),
          pltpu.SemaphoreType.DMA,
      ],
  )(x0, x1, table_blocks)


def kernel(x, table):
  x0 = jnp.asarray(x[:, 0], jnp.int32)
  x1 = jnp.asarray(x[:, 1], jnp.int32)
  table_blocks = table.reshape(table.shape[0] // 8, 8 * _D)
  return _run(x0, x1, table_blocks).reshape(_B, 1)


# speedup vs baseline: 1.0003x; 1.0003x over previous
import functools

import jax
import jax.numpy as jnp
from jax import lax
from jax.experimental import pallas as pl
from jax.experimental.pallas import tpu as pltpu
from jax.experimental.pallas import tpu_sc as plsc

_FIELD0 = 1000000
_BLK1 = _FIELD0 // 8
_B = 16384
_D = 16
_NC, _NS = 2, 16
_NW = _NC * _NS
_BPW = _B // _NW
_CH = 128
_NCH = _BPW // _CH
_GPC = _CH // _D


def _body(x0_hbm, x1_hbm, table_hbm, out_hbm,
          xi0_v, xi1_v, blk0_v, blk1_v, dat0_v, dat1_v, out_v, sem):
  wid = lax.axis_index("s") * _NC + lax.axis_index("c")
  base = wid * _BPW

  pltpu.sync_copy(x0_hbm.at[pl.ds(base, _BPW)], xi0_v)
  pltpu.sync_copy(x1_hbm.at[pl.ds(base, _BPW)], xi1_v)
  for c in range(_BPW // _D):
    j, sl = c // _GPC, pl.ds((c % _GPC) * _D, _D)
    blk0_v[j, sl] = lax.shift_right_logical(xi0_v[pl.ds(c * _D, _D)], 3)
    blk1_v[j, sl] = lax.shift_right_logical(xi1_v[pl.ds(c * _D, _D)], 3) + _BLK1

  lane = lax.iota(jnp.int32, _D)

  for j in range(_NCH):
    c0 = pltpu.async_copy(table_hbm.at[blk0_v.at[j]], dat0_v, sem)
    c1 = pltpu.async_copy(table_hbm.at[blk1_v.at[j]], dat1_v, sem)
    c0.wait()
    c1.wait()
    for g in range(_GPC):
      row = pl.ds((j * _GPC + g) * _D, _D)
      off0 = (xi0_v[row] & 7) * _D
      off1 = (xi1_v[row] & 7) * _D
      gr = g * _D + lane
      acc = jnp.zeros((_D,), jnp.float32)
      for l in range(_D):
        a = plsc.load_gather(dat0_v, [gr, off0 + l])
        b = plsc.load_gather(dat1_v, [gr, off1 + l])
        acc = acc + a * b
      e = jnp.exp(-jnp.abs(acc))
      out_v[row] = jnp.where(acc >= 0.0, 1.0 / (1.0 + e), e / (1.0 + e))

  pltpu.sync_copy(out_v, out_hbm.at[pl.ds(base, _BPW)])


@jax.jit
def _run(x0, x1, table_blocks):
  mesh = plsc.VectorSubcoreMesh(core_axis_name="c", subcore_axis_name="s",
                                num_cores=_NC, num_subcores=_NS)
  return pl.kernel(
      _body,
      out_type=jax.ShapeDtypeStruct((_B,), jnp.float32),
      mesh=mesh,
      compiler_params=pltpu.CompilerParams(needs_layout_passes=False,
                                           use_tc_tiling_on_sc=True),
      scratch_types=[
          pltpu.VMEM((_BPW,), jnp.int32),
          pltpu.VMEM((_BPW,), jnp.int32),
          pltpu.VMEM((_NCH, _CH), jnp.int32),
          pltpu.VMEM((_NCH, _CH), jnp.int32),
          pltpu.VMEM((_CH, 8 * _D), jnp.float32),
          pltpu.VMEM((_CH, 8 * _D), jnp.float32),
          pltpu.VMEM((_BPW,), jnp.float32),
          pltpu.SemaphoreType.DMA,
      ],
  )(x0, x1, table_blocks)


def kernel(x, table):
  x0 = jnp.asarray(x[:, 0], jnp.int32)
  x1 = jnp.asarray(x[:, 1], jnp.int32)
  table_blocks = table.reshape(table.shape[0] // 8, 8 * _D)
  return _run(x0, x1, table_blocks).reshape(_B, 1)


# speedup vs baseline: 21.4110x; 21.4045x over previous
import jax
import jax.numpy as jnp
from jax import lax
from jax.experimental import pallas as pl
from jax.experimental.pallas import tpu as pltpu
from jax.experimental.pallas import tpu_sc as plsc

_FIELD0 = 1000000
_B = 16384
_D = 16
_NC, _NS = 2, 16
_NW = _NC * _NS
_BPW = _B // _NW
_NG = _BPW // _D
_EPW = _BPW * 2 * _D
_CH = 128
_NCH = _EPW // _CH
_SLAB = 8 * _FIELD0 * 2


def _body(x0_hbm, x1_hbm, tflat_hbm, out_hbm, xi0_v, xi1_v, idx_v, dat_v,
          out_v, sem):
  wid = lax.axis_index("s") * _NC + lax.axis_index("c")
  base = wid * _BPW

  pltpu.sync_copy(x0_hbm.at[pl.ds(base, _BPW)], xi0_v)
  pltpu.sync_copy(x1_hbm.at[pl.ds(base, _BPW)], xi1_v)

  def build(g, carry):
    row = pl.ds(g * _D, _D)
    for f, ref, off in ((0, xi0_v, 0), (1, xi1_v, _FIELD0)):
      iv = ref[row] + off
      ebase = (lax.shift_right_logical(iv, 7) << 10) + (iv & 127)
      for j in range(_D):
        s0 = f * _D + j
        jo = (j >> 3) * _SLAB + (j & 7) * _CH
        idx_v[pl.ds((g * 2 * _D + s0) * _D, _D)] = ebase + jo
    return carry

  lax.fori_loop(0, _NG, build, 0)

  def fire(ch, carry):
    sl = pl.ds(ch * _CH, _CH)
    pltpu.async_copy(tflat_hbm.at[idx_v.at[sl]], dat_v.at[sl], sem)
    return carry

  lax.fori_loop(0, _NCH, fire, 0)
  pltpu.make_async_copy(tflat_hbm.at[pl.ds(0, _EPW)], dat_v, sem).wait()

  def dot(g, carry):
    eb = g * 2 * _D * _D
    acc = jnp.zeros((_D,), jnp.float32)
    for j in range(_D):
      a = dat_v[pl.ds(eb + j * _D, _D)]
      b = dat_v[pl.ds(eb + (_D + j) * _D, _D)]
      acc = acc + a * b
    e = jnp.exp(-jnp.abs(acc))
    out_v[pl.ds(g * _D, _D)] = jnp.where(acc >= 0.0, 1.0 / (1.0 + e),
                                         e / (1.0 + e))
    return carry

  lax.fori_loop(0, _NG, dot, 0)

  pltpu.sync_copy(out_v, out_hbm.at[pl.ds(base, _BPW)])


@jax.jit
def _run(x0, x1, tflat):
  mesh = plsc.VectorSubcoreMesh(core_axis_name="c", subcore_axis_name="s",
                                num_cores=_NC, num_subcores=_NS)
  return pl.kernel(
      _body,
      out_type=jax.ShapeDtypeStruct((_B,), jnp.float32),
      mesh=mesh,
      compiler_params=pltpu.CompilerParams(needs_layout_passes=False,
                                           use_tc_tiling_on_sc=False),
      scratch_types=[
          pltpu.VMEM((_BPW,), jnp.int32),
          pltpu.VMEM((_BPW,), jnp.int32),
          pltpu.VMEM((_EPW,), jnp.int32),
          pltpu.VMEM((_EPW,), jnp.float32),
          pltpu.VMEM((_BPW,), jnp.float32),
          pltpu.SemaphoreType.DMA,
      ],
  )(x0, x1, tflat)


def kernel(x, table):
  x0 = jnp.asarray(x[:, 0], jnp.int32)
  x1 = jnp.asarray(x[:, 1], jnp.int32)
  n = table.shape[0]
  tflat = (table.T.reshape(2, 8, n // 128, 128)
           .transpose(0, 2, 1, 3).reshape(n * _D))
  return _run(x0, x1, tflat).reshape(_B, 1)
